# baseline (speedup 1.0000x reference)
import functools

import jax
import jax.numpy as jnp
from jax import lax
from jax.experimental import pallas as pl
from jax.experimental.pallas import tpu as pltpu
from jax.experimental.pallas import tpu_sc as plsc

VOCAB = 1000000
DIM = 64
BATCH = 4096
SEQ = 200

N = BATCH * SEQ
NUM_WORKERS = 32
PER_W = N // NUM_WORKERS
CHUNK = 512
NCHUNKS = PER_W // CHUNK


def _emb_body(idx_hbm, table_hbm, out_hbm, idx_v, rows_v, gsem):
    wid = lax.axis_index("s") * 2 + lax.axis_index("c")
    base = wid * PER_W
    pltpu.sync_copy(idx_hbm.at[pl.ds(base, PER_W)], idx_v)

    @pl.loop(0, NCHUNKS)
    def _chunk(g):
        off = g * CHUNK
        idx_slice = idx_v.at[pl.ds(off, CHUNK)]
        pltpu.async_copy(table_hbm.at[idx_slice], rows_v, gsem).wait()
        pltpu.sync_copy(rows_v, out_hbm.at[pl.ds(base + off, CHUNK)])


@jax.jit
def _embed(x_flat, table):
    mesh = plsc.VectorSubcoreMesh(core_axis_name="c", subcore_axis_name="s")
    f = functools.partial(
        pl.kernel,
        out_type=jax.ShapeDtypeStruct((N, DIM), jnp.float32),
        mesh=mesh,
        scratch_types=[
            pltpu.VMEM((PER_W,), jnp.int32),
            pltpu.VMEM((CHUNK, DIM), jnp.float32),
            pltpu.SemaphoreType.DMA,
        ],
        compiler_params=pltpu.CompilerParams(use_tc_tiling_on_sc=False),
    )(_emb_body)
    return f(x_flat, table)


def kernel(X, table):
    out = _embed(X.reshape(-1), table)
    return out.reshape(BATCH, SEQ, DIM)


# speedup vs baseline: 1.0240x; 1.0240x over previous
import functools

import jax
import jax.numpy as jnp
from jax import lax
from jax.experimental import pallas as pl
from jax.experimental.pallas import tpu as pltpu
from jax.experimental.pallas import tpu_sc as plsc

VOCAB = 1000000
DIM = 64
BATCH = 4096
SEQ = 200

N = BATCH * SEQ
NUM_WORKERS = 32
PER_W = N // NUM_WORKERS
CHUNK = 512
NCHUNKS = PER_W // CHUNK


def _emb_body(idx_hbm, table_hbm, out_hbm, idx_v, rows0, rows1,
              gsem0, gsem1, wsem0, wsem1):
    wid = lax.axis_index("s") * 2 + lax.axis_index("c")
    base = wid * PER_W
    pltpu.sync_copy(idx_hbm.at[pl.ds(base, PER_W)], idx_v)

    rows = (rows0, rows1)
    gsems = (gsem0, gsem1)
    wsems = (wsem0, wsem1)

    def start_gather(g, b):
        idx_slice = idx_v.at[pl.ds(g * CHUNK, CHUNK)]
        pltpu.async_copy(table_hbm.at[idx_slice], rows[b], gsems[b])

    def wait_gather(b):
        pltpu.make_async_copy(
            table_hbm.at[idx_v.at[pl.ds(0, CHUNK)]], rows[b], gsems[b]).wait()

    def start_write(g, b):
        pltpu.async_copy(rows[b], out_hbm.at[pl.ds(base + g * CHUNK, CHUNK)],
                         wsems[b])

    def wait_write(b):
        pltpu.make_async_copy(
            rows[b], out_hbm.at[pl.ds(base, CHUNK)], wsems[b]).wait()

    def half(g, a, bb):
        @pl.when(g >= 1)
        def _():
            wait_write(bb)

        @pl.when(g + 1 < NCHUNKS)
        def _():
            start_gather(g + 1, bb)
        wait_gather(a)
        start_write(g, a)

    start_gather(0, 0)

    @pl.loop(0, NCHUNKS, step=2)
    def _chunk(g):
        half(g, 0, 1)
        half(g + 1, 1, 0)

    wait_write(1)


@jax.jit
def _embed(x_flat, table):
    mesh = plsc.VectorSubcoreMesh(core_axis_name="c", subcore_axis_name="s")
    f = functools.partial(
        pl.kernel,
        out_type=jax.ShapeDtypeStruct((N, DIM), jnp.float32),
        mesh=mesh,
        scratch_types=[
            pltpu.VMEM((PER_W,), jnp.int32),
            pltpu.VMEM((CHUNK, DIM), jnp.float32),
            pltpu.VMEM((CHUNK, DIM), jnp.float32),
            pltpu.SemaphoreType.DMA,
            pltpu.SemaphoreType.DMA,
            pltpu.SemaphoreType.DMA,
            pltpu.SemaphoreType.DMA,
        ],
        compiler_params=pltpu.CompilerParams(use_tc_tiling_on_sc=False),
    )(_emb_body)
    return f(x_flat, table)


def kernel(X, table):
    out = _embed(X.reshape(-1), table)
    return out.reshape(BATCH, SEQ, DIM)
